# Initial kernel scaffold; baseline (speedup 1.0000x reference)
import jax
import jax.numpy as jnp
from jax.experimental import pallas as pl


def kernel(x):
    raise NotImplementedError("write your pallas kernel here")


# speedup vs baseline: 1.3510x; 1.3510x over previous
import functools

import jax
import jax.numpy as jnp
import numpy as np
from jax.experimental import pallas as pl
from jax.experimental.pallas import tpu as pltpu

FH, FW = 32, 32
GH, GW = 16, 16
CH = 768
MERGE = 128
NTOK = FH * FW
NG = GH * GW
NOUT = MERGE + 4 * (NG - MERGE)


def _token_table():
    idx = np.arange(FH * FW)
    idx = idx.reshape(GH, 2, GW, 2).transpose(1, 3, 0, 2).reshape(4, GH * GW)
    return jnp.asarray(idx, dtype=jnp.int32)


_TOK2 = _token_table()


def _gather_body(tok_ref, x_ref, o_ref):
    tok = tok_ref[0]
    x = x_ref[0]
    lane = jax.lax.broadcasted_iota(jnp.int32, (NOUT, NTOK), 1)
    row = jax.lax.broadcasted_iota(jnp.int32, (NOUT, 1), 0)
    is_m = row < MERGE
    base = jnp.where(is_m, 64 * (tok >> 4) + 2 * (tok & 15), tok)
    hit = lane == base
    hit = hit | (is_m & ((lane == base + 1) | (lane == base + 32) | (lane == base + 33)))
    w = jnp.where(is_m, jnp.float32(0.25), jnp.float32(1.0))
    s = jnp.where(hit, w, jnp.float32(0.0)).astype(jnp.bfloat16)
    xb = x.astype(jnp.bfloat16)
    o_ref[0] = jax.lax.dot_general(
        s, xb, (((1,), (1,)), ((), ())), preferred_element_type=jnp.float32)


@jax.jit
def _gather_pallas(x_flat, tok3):
    B = x_flat.shape[0]
    return pl.pallas_call(
        _gather_body,
        grid=(B,),
        in_specs=[
            pl.BlockSpec((1, NOUT, 1), lambda b: (b, 0, 0)),
            pl.BlockSpec((1, CH, NTOK), lambda b: (b, 0, 0)),
        ],
        out_specs=pl.BlockSpec((1, NOUT, CH), lambda b: (b, 0, 0)),
        out_shape=jax.ShapeDtypeStruct((B, NOUT, CH), jnp.float32),
    )(tok3, x_flat)


def kernel(x):
    B, C, H, W = x.shape
    xr = x[:, :, 0::2, 0::2]
    xrf = jnp.broadcast_to(
        xr[:, :, :, None, :, None], (B, C, GH, 2, GW, 2)).reshape(B, C, H, W)
    eps = 1e-8
    dot = jnp.sum(xrf * x, axis=1)
    n1 = jnp.sqrt(jnp.sum(xrf * xrf, axis=1))
    n2 = jnp.sqrt(jnp.sum(x * x, axis=1))
    cos = dot / jnp.maximum(n1 * n2, eps)
    score = cos.reshape(B, GH, 2, GW, 2).mean(axis=(2, 4)).reshape(B, NG)
    order = jnp.argsort(score, axis=1, stable=True)
    idx_merge = order[:, :MERGE]
    idx_keep = order[:, MERGE:]
    keep_tok = jnp.take(_TOK2, idx_keep, axis=1).transpose(1, 0, 2).reshape(B, -1)
    token_idx = jnp.concatenate([idx_merge, keep_tok], axis=1)
    aligned = jnp.concatenate([_TOK2[0][idx_merge], keep_tok], axis=1)
    x_out = _gather_pallas(
        x.reshape(B, C, H * W),
        token_idx.reshape(B, NOUT, 1).astype(jnp.int32))
    return x_out, aligned


# speedup vs baseline: 1.4061x; 1.0408x over previous
import functools

import jax
import jax.numpy as jnp
import numpy as np
from jax import lax
from jax.experimental import pallas as pl
from jax.experimental.pallas import tpu as pltpu
from jax.experimental.pallas import tpu_sc as plsc

FH, FW = 32, 32
GH, GW = 16, 16
CH = 768
MERGE = 128
NTOK = FH * FW
NG = GH * GW
NOUT = MERGE + 4 * (NG - MERGE)


def _token_table():
    idx = np.arange(FH * FW)
    idx = idx.reshape(GH, 2, GW, 2).transpose(1, 3, 0, 2).reshape(4, GH * GW)
    return idx.astype(np.int32)


_TOK2_NP = _token_table()


def _sc_sort_body(score_hbm, tok_hbm, al_hbm, svm, ovm, tvm, avm):
    wid = lax.axis_index("s") * 2 + lax.axis_index("c")
    pltpu.sync_copy(score_hbm.at[wid], svm)
    iota16 = lax.iota(jnp.int32, 16)
    keys = [svm[pl.ds(16 * ic, 16)] for ic in range(16)]
    zero = jnp.zeros((16,), jnp.int32)

    def body(j, cnts):
        kj = plsc.load_gather(svm, [jnp.full((16,), j, jnp.int32)])
        out = []
        for ic in range(16):
            lt = (kj < keys[ic]).astype(jnp.int32)
            tie = ((kj == keys[ic]) & (j < 16 * ic + iota16)).astype(jnp.int32)
            out.append(cnts[ic] + lt + tie)
        return tuple(out)

    ranks = lax.fori_loop(0, 256, body, (zero,) * 16)
    for ic in range(16):
        plsc.store_scatter(ovm, [ranks[ic]], 16 * ic + iota16)

    for ic in range(8):
        g = ovm[pl.ds(16 * ic, 16)]
        tvm[pl.ds(16 * ic, 16)] = g
        avm[pl.ds(16 * ic, 16)] = ((g >> 4) << 6) + ((g & 15) << 1)
    for ic in range(8):
        g = ovm[pl.ds(128 + 16 * ic, 16)]
        t00 = ((g >> 4) << 6) + ((g & 15) << 1)
        for p, off in enumerate((0, 1, 32, 33)):
            dst = 128 + 128 * p + 16 * ic
            tvm[pl.ds(dst, 16)] = t00 + off
            avm[pl.ds(dst, 16)] = t00 + off
    pltpu.sync_copy(tvm, tok_hbm.at[wid])
    pltpu.sync_copy(avm, al_hbm.at[wid])


@jax.jit
def _sc_sort(score):
    B = score.shape[0]
    mesh = plsc.VectorSubcoreMesh(core_axis_name="c", subcore_axis_name="s")
    f = functools.partial(
        pl.kernel,
        out_type=[jax.ShapeDtypeStruct((B, NOUT), jnp.int32),
                  jax.ShapeDtypeStruct((B, NOUT), jnp.int32)],
        mesh=mesh,
        compiler_params=pltpu.CompilerParams(needs_layout_passes=False),
        scratch_types=[
            pltpu.VMEM((NG,), jnp.float32),
            pltpu.VMEM((NG,), jnp.int32),
            pltpu.VMEM((NOUT,), jnp.int32),
            pltpu.VMEM((NOUT,), jnp.int32),
        ],
    )(_sc_sort_body)
    return f(score)


def _gather_body(tok_ref, x_ref, o_ref):
    tok = tok_ref[0]
    x = x_ref[0]
    lane = jax.lax.broadcasted_iota(jnp.int32, (NOUT, NTOK), 1)
    row = jax.lax.broadcasted_iota(jnp.int32, (NOUT, 1), 0)
    is_m = row < MERGE
    base = jnp.where(is_m, 64 * (tok >> 4) + 2 * (tok & 15), tok)
    hit = lane == base
    hit = hit | (is_m & ((lane == base + 1) | (lane == base + 32) | (lane == base + 33)))
    w = jnp.where(is_m, jnp.float32(0.25), jnp.float32(1.0))
    s = jnp.where(hit, w, jnp.float32(0.0)).astype(jnp.bfloat16)
    xb = x.astype(jnp.bfloat16)
    o_ref[0] = jax.lax.dot_general(
        s, xb, (((1,), (1,)), ((), ())), preferred_element_type=jnp.float32)


@jax.jit
def _gather_pallas(x_flat, tok3):
    B = x_flat.shape[0]
    return pl.pallas_call(
        _gather_body,
        grid=(B,),
        in_specs=[
            pl.BlockSpec((1, NOUT, 1), lambda b: (b, 0, 0)),
            pl.BlockSpec((1, CH, NTOK), lambda b: (b, 0, 0)),
        ],
        out_specs=pl.BlockSpec((1, NOUT, CH), lambda b: (b, 0, 0)),
        out_shape=jax.ShapeDtypeStruct((B, NOUT, CH), jnp.float32),
    )(tok3, x_flat)


def kernel(x):
    B, C, H, W = x.shape
    xr = x[:, :, 0::2, 0::2]
    xrf = jnp.broadcast_to(
        xr[:, :, :, None, :, None], (B, C, GH, 2, GW, 2)).reshape(B, C, H, W)
    dot = jnp.sum(xrf * x, axis=1)
    n1 = jnp.sqrt(jnp.sum(xrf * xrf, axis=1))
    n2 = jnp.sqrt(jnp.sum(x * x, axis=1))
    cos = dot / jnp.maximum(n1 * n2, 1e-8)
    score = cos.reshape(B, GH, 2, GW, 2).mean(axis=(2, 4)).reshape(B, NG)
    token_idx, aligned = _sc_sort(score)
    x_out = _gather_pallas(
        x.reshape(B, C, H * W), token_idx.reshape(B, NOUT, 1))
    return x_out, aligned


# speedup vs baseline: 1.4185x; 1.0088x over previous
import functools

import jax
import jax.numpy as jnp
import numpy as np
from jax import lax
from jax.experimental import pallas as pl
from jax.experimental.pallas import tpu as pltpu
from jax.experimental.pallas import tpu_sc as plsc

FH, FW = 32, 32
GH, GW = 16, 16
CH = 768
MERGE = 128
NTOK = FH * FW
NG = GH * GW
NOUT = MERGE + 4 * (NG - MERGE)


def _token_table():
    idx = np.arange(FH * FW)
    idx = idx.reshape(GH, 2, GW, 2).transpose(1, 3, 0, 2).reshape(4, GH * GW)
    return idx.astype(np.int32)


_TOK2_NP = _token_table()


def _sc_sort_body(score_hbm, tok_hbm, al_hbm, svm, ovm, tvm, avm):
    wid = lax.axis_index("s") * 2 + lax.axis_index("c")
    pltpu.sync_copy(score_hbm.at[wid], svm)
    iota16 = lax.iota(jnp.int32, 16)
    keys = [svm[pl.ds(16 * ic, 16)] for ic in range(16)]
    zero = jnp.zeros((16,), jnp.int32)

    def body(j, cnts):
        kj = plsc.load_gather(svm, [jnp.full((16,), j, jnp.int32)])
        out = []
        for ic in range(16):
            lt = (kj < keys[ic]).astype(jnp.int32)
            tie = ((kj == keys[ic]) & (j < 16 * ic + iota16)).astype(jnp.int32)
            out.append(cnts[ic] + lt + tie)
        return tuple(out)

    ranks = lax.fori_loop(0, 256, body, (zero,) * 16)
    for ic in range(16):
        plsc.store_scatter(ovm, [ranks[ic]], 16 * ic + iota16)

    for ic in range(8):
        g = ovm[pl.ds(16 * ic, 16)]
        tvm[pl.ds(16 * ic, 16)] = g
        avm[pl.ds(16 * ic, 16)] = ((g >> 4) << 6) + ((g & 15) << 1)
    for ic in range(8):
        g = ovm[pl.ds(128 + 16 * ic, 16)]
        t00 = ((g >> 4) << 6) + ((g & 15) << 1)
        for p, off in enumerate((0, 1, 32, 33)):
            dst = 128 + 128 * p + 16 * ic
            tvm[pl.ds(dst, 16)] = t00 + off
            avm[pl.ds(dst, 16)] = t00 + off
    pltpu.sync_copy(tvm, tok_hbm.at[wid])
    pltpu.sync_copy(avm, al_hbm.at[wid])


@jax.jit
def _sc_sort(score):
    B = score.shape[0]
    mesh = plsc.VectorSubcoreMesh(core_axis_name="c", subcore_axis_name="s")
    f = functools.partial(
        pl.kernel,
        out_type=[jax.ShapeDtypeStruct((B, NOUT), jnp.int32),
                  jax.ShapeDtypeStruct((B, NOUT), jnp.int32)],
        mesh=mesh,
        compiler_params=pltpu.CompilerParams(needs_layout_passes=False),
        scratch_types=[
            pltpu.VMEM((NG,), jnp.float32),
            pltpu.VMEM((NG,), jnp.int32),
            pltpu.VMEM((NOUT,), jnp.int32),
            pltpu.VMEM((NOUT,), jnp.int32),
        ],
    )(_sc_sort_body)
    return f(score)


def _gather_body(tok_ref, x_ref, o_ref):
    tok = tok_ref[0]
    x = x_ref[0]
    lane = jax.lax.broadcasted_iota(jnp.int32, (NOUT, NTOK), 1)
    row = jax.lax.broadcasted_iota(jnp.int32, (NOUT, 1), 0)
    is_m = row < MERGE
    base = jnp.where(is_m, 64 * (tok >> 4) + 2 * (tok & 15), tok)
    hit = lane == base
    hit = hit | (is_m & ((lane == base + 1) | (lane == base + 32) | (lane == base + 33)))
    w = jnp.where(is_m, jnp.float32(0.25), jnp.float32(1.0))
    s = jnp.where(hit, w, jnp.float32(0.0)).astype(jnp.bfloat16)
    xb = x.astype(jnp.bfloat16)
    o_ref[0] = jax.lax.dot_general(
        s, xb, (((1,), (1,)), ((), ())), preferred_element_type=jnp.float32)


@jax.jit
def _gather_pallas(x_flat, tok3):
    B = x_flat.shape[0]
    return pl.pallas_call(
        _gather_body,
        grid=(B,),
        in_specs=[
            pl.BlockSpec((1, NOUT, 1), lambda b: (b, 0, 0)),
            pl.BlockSpec((1, CH, NTOK), lambda b: (b, 0, 0)),
        ],
        out_specs=pl.BlockSpec((1, NOUT, CH), lambda b: (b, 0, 0)),
        out_shape=jax.ShapeDtypeStruct((B, NOUT, CH), jnp.float32),
    )(tok3, x_flat)


def kernel(x):
    B, C, H, W = x.shape
    xr = x[:, :, 0::2, 0::2]
    xrf = jnp.broadcast_to(
        xr[:, :, :, None, :, None], (B, C, GH, 2, GW, 2)).reshape(B, C, H, W)
    dot = jnp.sum(xrf * x, axis=1)
    nn = jnp.sum(x * x, axis=1)
    nn_up = jnp.broadcast_to(
        nn[:, 0::2, None, 0::2, None], (B, GH, 2, GW, 2)).reshape(B, H, W)
    cos = dot / jnp.maximum(jnp.sqrt(nn_up) * jnp.sqrt(nn), 1e-8)
    score = cos.reshape(B, GH, 2, GW, 2).mean(axis=(2, 4)).reshape(B, NG)
    token_idx, aligned = _sc_sort(score)
    x_out = _gather_pallas(
        x.reshape(B, C, H * W), token_idx.reshape(B, NOUT, 1))
    return x_out, aligned


# speedup vs baseline: 5.8658x; 4.1352x over previous
import functools

import jax
import jax.numpy as jnp
import numpy as np
from jax import lax
from jax.experimental import pallas as pl
from jax.experimental.pallas import tpu as pltpu
from jax.experimental.pallas import tpu_sc as plsc

FH, FW = 32, 32
GH, GW = 16, 16
CH = 768
MERGE = 128
NTOK = FH * FW
NG = GH * GW
NOUT = MERGE + 4 * (NG - MERGE)


def _token_table():
    idx = np.arange(FH * FW)
    idx = idx.reshape(GH, 2, GW, 2).transpose(1, 3, 0, 2).reshape(4, GH * GW)
    return idx.astype(np.int32)


_TOK2_NP = _token_table()


def _score_body(x_ref, s_ref):
    xt = x_ref[0]
    row = jax.lax.broadcasted_iota(jnp.int32, (NTOK, CH), 0)
    m1 = (row & 1) != 0
    m32 = (row & 32) != 0
    mb = m1 & m32
    r1 = jnp.roll(xt, 1, axis=0)
    r32 = jnp.roll(xt, 32, axis=0)
    r33 = jnp.roll(xt, 33, axis=0)
    xref = jnp.where(mb, r33, jnp.where(m32, r32, jnp.where(m1, r1, xt)))

    def chan_reduce(p):
        a0 = p[:, 0:128] + p[:, 128:256]
        a1 = p[:, 256:384] + p[:, 384:512]
        a2 = p[:, 512:640] + p[:, 640:768]
        r0 = jnp.sum(a0, axis=1, keepdims=True)
        r1_ = jnp.sum(a1, axis=1, keepdims=True)
        r2 = jnp.sum(a2, axis=1, keepdims=True)
        return (r0 + r1_) + r2

    dot = chan_reduce(xref * xt)
    nn = chan_reduce(xt * xt)
    rowc = jax.lax.broadcasted_iota(jnp.int32, (NTOK, 1), 0)
    c1 = (rowc & 1) != 0
    c32 = (rowc & 32) != 0
    cb = c1 & c32
    nn_ref = jnp.where(cb, jnp.roll(nn, 33, axis=0),
                       jnp.where(c32, jnp.roll(nn, 32, axis=0),
                                 jnp.where(c1, jnp.roll(nn, 1, axis=0), nn)))
    denom = jnp.maximum(jnp.sqrt(nn_ref) * jnp.sqrt(nn), jnp.float32(1e-8))
    cos = dot / denom
    pooled = ((cos + jnp.roll(cos, -1, axis=0)) + jnp.roll(cos, -32, axis=0)
              ) + jnp.roll(cos, -33, axis=0)
    s_ref[0] = pooled / 4.0


@jax.jit
def _score_pallas(xt3):
    B = xt3.shape[0]
    return pl.pallas_call(
        _score_body,
        grid=(B,),
        in_specs=[pl.BlockSpec((1, NTOK, CH), lambda b: (b, 0, 0))],
        out_specs=pl.BlockSpec((1, NTOK, 1), lambda b: (b, 0, 0)),
        out_shape=jax.ShapeDtypeStruct((B, NTOK, 1), jnp.float32),
    )(xt3)


def _sc_sort_body(score_hbm, tok_hbm, al_hbm, svm, s256, ovm, tvm, avm):
    wid = lax.axis_index("s") * 2 + lax.axis_index("c")
    pltpu.sync_copy(score_hbm.at[wid], svm)
    iota16 = lax.iota(jnp.int32, 16)
    for ic in range(16):
        g = 16 * ic + iota16
        t00 = ((g >> 4) << 6) + ((g & 15) << 1)
        s256[pl.ds(16 * ic, 16)] = plsc.load_gather(svm, [t00])
    keys = [s256[pl.ds(16 * ic, 16)] for ic in range(16)]
    zero = jnp.zeros((16,), jnp.int32)

    def body(j, cnts):
        kj = plsc.load_gather(s256, [jnp.full((16,), j, jnp.int32)])
        out = []
        for ic in range(16):
            lt = (kj < keys[ic]).astype(jnp.int32)
            tie = ((kj == keys[ic]) & (j < 16 * ic + iota16)).astype(jnp.int32)
            out.append(cnts[ic] + lt + tie)
        return tuple(out)

    ranks = lax.fori_loop(0, 256, body, (zero,) * 16)
    for ic in range(16):
        plsc.store_scatter(ovm, [ranks[ic]], 16 * ic + iota16)

    for ic in range(8):
        g = ovm[pl.ds(16 * ic, 16)]
        tvm[pl.ds(16 * ic, 16)] = g
        avm[pl.ds(16 * ic, 16)] = ((g >> 4) << 6) + ((g & 15) << 1)
    for ic in range(8):
        g = ovm[pl.ds(128 + 16 * ic, 16)]
        t00 = ((g >> 4) << 6) + ((g & 15) << 1)
        for p, off in enumerate((0, 1, 32, 33)):
            dst = 128 + 128 * p + 16 * ic
            tvm[pl.ds(dst, 16)] = t00 + off
            avm[pl.ds(dst, 16)] = t00 + off
    pltpu.sync_copy(tvm, tok_hbm.at[wid])
    pltpu.sync_copy(avm, al_hbm.at[wid])


@jax.jit
def _sc_sort(score):
    B = score.shape[0]
    mesh = plsc.VectorSubcoreMesh(core_axis_name="c", subcore_axis_name="s")
    f = functools.partial(
        pl.kernel,
        out_type=[jax.ShapeDtypeStruct((B, NOUT), jnp.int32),
                  jax.ShapeDtypeStruct((B, NOUT), jnp.int32)],
        mesh=mesh,
        compiler_params=pltpu.CompilerParams(needs_layout_passes=False),
        scratch_types=[
            pltpu.VMEM((NTOK,), jnp.float32),
            pltpu.VMEM((NG,), jnp.float32),
            pltpu.VMEM((NG,), jnp.int32),
            pltpu.VMEM((NOUT,), jnp.int32),
            pltpu.VMEM((NOUT,), jnp.int32),
        ],
    )(_sc_sort_body)
    return f(score)


def _gather_body(tok_ref, x_ref, o_ref):
    tok = tok_ref[0]
    x = x_ref[0]
    lane = jax.lax.broadcasted_iota(jnp.int32, (NOUT, NTOK), 1)
    row = jax.lax.broadcasted_iota(jnp.int32, (NOUT, 1), 0)
    is_m = row < MERGE
    base = jnp.where(is_m, 64 * (tok >> 4) + 2 * (tok & 15), tok)
    hit = lane == base
    hit = hit | (is_m & ((lane == base + 1) | (lane == base + 32) | (lane == base + 33)))
    w = jnp.where(is_m, jnp.float32(0.25), jnp.float32(1.0))
    s = jnp.where(hit, w, jnp.float32(0.0)).astype(jnp.bfloat16)
    xb = x.astype(jnp.bfloat16)
    o_ref[0] = jax.lax.dot_general(
        s, xb, (((1,), (0,)), ((), ())), preferred_element_type=jnp.float32)


@jax.jit
def _gather_pallas(x_flat, tok3):
    B = x_flat.shape[0]
    return pl.pallas_call(
        _gather_body,
        grid=(B,),
        in_specs=[
            pl.BlockSpec((1, NOUT, 1), lambda b: (b, 0, 0)),
            pl.BlockSpec((1, NTOK, CH), lambda b: (b, 0, 0)),
        ],
        out_specs=pl.BlockSpec((1, NOUT, CH), lambda b: (b, 0, 0)),
        out_shape=jax.ShapeDtypeStruct((B, NOUT, CH), jnp.float32),
    )(tok3, x_flat)


def kernel(x):
    B, C, H, W = x.shape
    xt3 = jnp.transpose(x, (0, 2, 3, 1)).reshape(B, H * W, C)
    smap = _score_pallas(xt3).reshape(B, NTOK)
    token_idx, aligned = _sc_sort(smap)
    x_out = _gather_pallas(xt3, token_idx.reshape(B, NOUT, 1))
    return x_out, aligned


# speedup vs baseline: 7.1154x; 1.2130x over previous
import functools

import jax
import jax.numpy as jnp
import numpy as np
from jax import lax
from jax.experimental import pallas as pl
from jax.experimental.pallas import tpu as pltpu
from jax.experimental.pallas import tpu_sc as plsc

FH, FW = 32, 32
GH, GW = 16, 16
CH = 768
MERGE = 128
NTOK = FH * FW
NG = GH * GW
NOUT = MERGE + 4 * (NG - MERGE)


def _token_table():
    idx = np.arange(FH * FW)
    idx = idx.reshape(GH, 2, GW, 2).transpose(1, 3, 0, 2).reshape(4, GH * GW)
    return idx.astype(np.int32)


_TOK2_NP = _token_table()


def _score_body(x_ref, s_ref):
    xt = x_ref[0]
    row = jax.lax.broadcasted_iota(jnp.int32, (NTOK, CH), 0)
    m1 = (row & 1) != 0
    m32 = (row & 32) != 0
    mb = m1 & m32
    r1 = jnp.roll(xt, 1, axis=0)
    r32 = jnp.roll(xt, 32, axis=0)
    r33 = jnp.roll(xt, 33, axis=0)
    xref = jnp.where(mb, r33, jnp.where(m32, r32, jnp.where(m1, r1, xt)))

    def chan_reduce(p):
        a0 = p[:, 0:128] + p[:, 128:256]
        a1 = p[:, 256:384] + p[:, 384:512]
        a2 = p[:, 512:640] + p[:, 640:768]
        r0 = jnp.sum(a0, axis=1, keepdims=True)
        r1_ = jnp.sum(a1, axis=1, keepdims=True)
        r2 = jnp.sum(a2, axis=1, keepdims=True)
        return (r0 + r1_) + r2

    dot = chan_reduce(xref * xt)
    nn = chan_reduce(xt * xt)
    rowc = jax.lax.broadcasted_iota(jnp.int32, (NTOK, 1), 0)
    c1 = (rowc & 1) != 0
    c32 = (rowc & 32) != 0
    cb = c1 & c32
    nn_ref = jnp.where(cb, jnp.roll(nn, 33, axis=0),
                       jnp.where(c32, jnp.roll(nn, 32, axis=0),
                                 jnp.where(c1, jnp.roll(nn, 1, axis=0), nn)))
    denom = jnp.maximum(jnp.sqrt(nn_ref) * jnp.sqrt(nn), jnp.float32(1e-8))
    cos = dot / denom
    pooled = ((cos + jnp.roll(cos, -1, axis=0)) + jnp.roll(cos, -32, axis=0)
              ) + jnp.roll(cos, -33, axis=0)
    s_ref[0] = pooled / 4.0


@jax.jit
def _score_pallas(xt3):
    B = xt3.shape[0]
    return pl.pallas_call(
        _score_body,
        grid=(B,),
        in_specs=[pl.BlockSpec((1, NTOK, CH), lambda b: (b, 0, 0))],
        out_specs=pl.BlockSpec((1, NTOK, 1), lambda b: (b, 0, 0)),
        out_shape=jax.ShapeDtypeStruct((B, NTOK, 1), jnp.float32),
    )(xt3)


def _sc_sort_body(score_hbm, tok_hbm, al_hbm, svm, s256, ovm, tvm, avm):
    wid = lax.axis_index("s") * 2 + lax.axis_index("c")
    pltpu.sync_copy(score_hbm.at[wid], svm)
    iota16 = lax.iota(jnp.int32, 16)
    for ic in range(16):
        g = 16 * ic + iota16
        t00 = ((g >> 4) << 6) + ((g & 15) << 1)
        s256[pl.ds(16 * ic, 16)] = plsc.load_gather(svm, [t00])
    keys = [s256[pl.ds(16 * ic, 16)] for ic in range(16)]
    zero = jnp.zeros((16,), jnp.int32)

    def body(jc, cnts):
        vj = plsc.load_gather(s256, [16 * jc + iota16])
        out = list(cnts)
        for l in range(16):
            kj = lax.gather(
                vj, jnp.full((16, 1), l, jnp.int32),
                lax.GatherDimensionNumbers(
                    offset_dims=(), collapsed_slice_dims=(0,),
                    start_index_map=(0,)),
                (1,), mode=lax.GatherScatterMode.PROMISE_IN_BOUNDS)
            jg = 16 * jc + l
            for ic in range(16):
                lt = (kj < keys[ic]).astype(jnp.int32)
                tie = ((kj == keys[ic]) & (jg < 16 * ic + iota16)
                       ).astype(jnp.int32)
                out[ic] = out[ic] + lt + tie
        return tuple(out)

    ranks = lax.fori_loop(0, 16, body, (zero,) * 16)
    for ic in range(16):
        plsc.store_scatter(ovm, [ranks[ic]], 16 * ic + iota16)

    for ic in range(8):
        g = ovm[pl.ds(16 * ic, 16)]
        tvm[pl.ds(16 * ic, 16)] = g
        avm[pl.ds(16 * ic, 16)] = ((g >> 4) << 6) + ((g & 15) << 1)
    for ic in range(8):
        g = ovm[pl.ds(128 + 16 * ic, 16)]
        t00 = ((g >> 4) << 6) + ((g & 15) << 1)
        for p, off in enumerate((0, 1, 32, 33)):
            dst = 128 + 128 * p + 16 * ic
            tvm[pl.ds(dst, 16)] = t00 + off
            avm[pl.ds(dst, 16)] = t00 + off
    pltpu.sync_copy(tvm, tok_hbm.at[wid])
    pltpu.sync_copy(avm, al_hbm.at[wid])


@jax.jit
def _sc_sort(score):
    B = score.shape[0]
    mesh = plsc.VectorSubcoreMesh(core_axis_name="c", subcore_axis_name="s")
    f = functools.partial(
        pl.kernel,
        out_type=[jax.ShapeDtypeStruct((B, NOUT), jnp.int32),
                  jax.ShapeDtypeStruct((B, NOUT), jnp.int32)],
        mesh=mesh,
        compiler_params=pltpu.CompilerParams(needs_layout_passes=False),
        scratch_types=[
            pltpu.VMEM((NTOK,), jnp.float32),
            pltpu.VMEM((NG,), jnp.float32),
            pltpu.VMEM((NG,), jnp.int32),
            pltpu.VMEM((NOUT,), jnp.int32),
            pltpu.VMEM((NOUT,), jnp.int32),
        ],
    )(_sc_sort_body)
    return f(score)


def _gather_body(tok_ref, x_ref, o_ref):
    tok = tok_ref[0]
    x = x_ref[0]
    lane = jax.lax.broadcasted_iota(jnp.int32, (NOUT, NTOK), 1)
    row = jax.lax.broadcasted_iota(jnp.int32, (NOUT, 1), 0)
    is_m = row < MERGE
    base = jnp.where(is_m, 64 * (tok >> 4) + 2 * (tok & 15), tok)
    hit = lane == base
    hit = hit | (is_m & ((lane == base + 1) | (lane == base + 32) | (lane == base + 33)))
    w = jnp.where(is_m, jnp.float32(0.25), jnp.float32(1.0))
    s = jnp.where(hit, w, jnp.float32(0.0)).astype(jnp.bfloat16)
    xb = x.astype(jnp.bfloat16)
    o_ref[0] = jax.lax.dot_general(
        s, xb, (((1,), (0,)), ((), ())), preferred_element_type=jnp.float32)


@jax.jit
def _gather_pallas(x_flat, tok3):
    B = x_flat.shape[0]
    return pl.pallas_call(
        _gather_body,
        grid=(B,),
        in_specs=[
            pl.BlockSpec((1, NOUT, 1), lambda b: (b, 0, 0)),
            pl.BlockSpec((1, NTOK, CH), lambda b: (b, 0, 0)),
        ],
        out_specs=pl.BlockSpec((1, NOUT, CH), lambda b: (b, 0, 0)),
        out_shape=jax.ShapeDtypeStruct((B, NOUT, CH), jnp.float32),
    )(tok3, x_flat)


def kernel(x):
    B, C, H, W = x.shape
    xt3 = jnp.transpose(x, (0, 2, 3, 1)).reshape(B, H * W, C)
    smap = _score_pallas(xt3).reshape(B, NTOK)
    token_idx, aligned = _sc_sort(smap)
    x_out = _gather_pallas(xt3, token_idx.reshape(B, NOUT, 1))
    return x_out, aligned


# speedup vs baseline: 7.1186x; 1.0004x over previous
import functools

import jax
import jax.numpy as jnp
import numpy as np
from jax import lax
from jax.experimental import pallas as pl
from jax.experimental.pallas import tpu as pltpu
from jax.experimental.pallas import tpu_sc as plsc

FH, FW = 32, 32
GH, GW = 16, 16
CH = 768
MERGE = 128
NTOK = FH * FW
NG = GH * GW
NOUT = MERGE + 4 * (NG - MERGE)


def _token_table():
    idx = np.arange(FH * FW)
    idx = idx.reshape(GH, 2, GW, 2).transpose(1, 3, 0, 2).reshape(4, GH * GW)
    return idx.astype(np.int32)


_TOK2_NP = _token_table()


def _score_body(x_ref, s_ref):
    xt = x_ref[0]
    row = jax.lax.broadcasted_iota(jnp.int32, (NTOK, CH), 0)
    m1 = (row & 1) != 0
    m32 = (row & 32) != 0
    mb = m1 & m32
    r1 = jnp.roll(xt, 1, axis=0)
    r32 = jnp.roll(xt, 32, axis=0)
    r33 = jnp.roll(xt, 33, axis=0)
    xref = jnp.where(mb, r33, jnp.where(m32, r32, jnp.where(m1, r1, xt)))

    def chan_reduce(p):
        a0 = p[:, 0:128] + p[:, 128:256]
        a1 = p[:, 256:384] + p[:, 384:512]
        a2 = p[:, 512:640] + p[:, 640:768]
        r0 = jnp.sum(a0, axis=1, keepdims=True)
        r1_ = jnp.sum(a1, axis=1, keepdims=True)
        r2 = jnp.sum(a2, axis=1, keepdims=True)
        return (r0 + r1_) + r2

    dot = chan_reduce(xref * xt)
    nn = chan_reduce(xt * xt)
    rowc = jax.lax.broadcasted_iota(jnp.int32, (NTOK, 1), 0)
    c1 = (rowc & 1) != 0
    c32 = (rowc & 32) != 0
    cb = c1 & c32
    nn_ref = jnp.where(cb, jnp.roll(nn, 33, axis=0),
                       jnp.where(c32, jnp.roll(nn, 32, axis=0),
                                 jnp.where(c1, jnp.roll(nn, 1, axis=0), nn)))
    denom = jnp.maximum(jnp.sqrt(nn_ref) * jnp.sqrt(nn), jnp.float32(1e-8))
    cos = dot / denom
    pooled = (cos + jnp.roll(cos, -32, axis=0)) + (
        jnp.roll(cos, -1, axis=0) + jnp.roll(cos, -33, axis=0))
    s_ref[0] = pooled / 4.0


@jax.jit
def _score_pallas(xt3):
    B = xt3.shape[0]
    return pl.pallas_call(
        _score_body,
        grid=(B,),
        in_specs=[pl.BlockSpec((1, NTOK, CH), lambda b: (b, 0, 0))],
        out_specs=pl.BlockSpec((1, NTOK, 1), lambda b: (b, 0, 0)),
        out_shape=jax.ShapeDtypeStruct((B, NTOK, 1), jnp.float32),
    )(xt3)


def _sc_sort_body(score_hbm, tok_hbm, al_hbm, svm, s256, ovm, tvm, avm):
    wid = lax.axis_index("s") * 2 + lax.axis_index("c")
    pltpu.sync_copy(score_hbm.at[wid], svm)
    iota16 = lax.iota(jnp.int32, 16)
    for ic in range(16):
        g = 16 * ic + iota16
        t00 = ((g >> 4) << 6) + ((g & 15) << 1)
        s256[pl.ds(16 * ic, 16)] = plsc.load_gather(svm, [t00])
    keys = [s256[pl.ds(16 * ic, 16)] for ic in range(16)]
    zero = jnp.zeros((16,), jnp.int32)

    def body(jc, cnts):
        vj = plsc.load_gather(s256, [16 * jc + iota16])
        out = list(cnts)
        for l in range(16):
            kj = lax.gather(
                vj, jnp.full((16, 1), l, jnp.int32),
                lax.GatherDimensionNumbers(
                    offset_dims=(), collapsed_slice_dims=(0,),
                    start_index_map=(0,)),
                (1,), mode=lax.GatherScatterMode.PROMISE_IN_BOUNDS)
            jg = 16 * jc + l
            for ic in range(16):
                lt = (kj < keys[ic]).astype(jnp.int32)
                tie = ((kj == keys[ic]) & (jg < 16 * ic + iota16)
                       ).astype(jnp.int32)
                out[ic] = out[ic] + lt + tie
        return tuple(out)

    ranks = lax.fori_loop(0, 16, body, (zero,) * 16)
    for ic in range(16):
        plsc.store_scatter(ovm, [ranks[ic]], 16 * ic + iota16)

    for ic in range(8):
        g = ovm[pl.ds(16 * ic, 16)]
        tvm[pl.ds(16 * ic, 16)] = g
        avm[pl.ds(16 * ic, 16)] = ((g >> 4) << 6) + ((g & 15) << 1)
    for ic in range(8):
        g = ovm[pl.ds(128 + 16 * ic, 16)]
        t00 = ((g >> 4) << 6) + ((g & 15) << 1)
        for p, off in enumerate((0, 1, 32, 33)):
            dst = 128 + 128 * p + 16 * ic
            tvm[pl.ds(dst, 16)] = t00 + off
            avm[pl.ds(dst, 16)] = t00 + off
    pltpu.sync_copy(tvm, tok_hbm.at[wid])
    pltpu.sync_copy(avm, al_hbm.at[wid])


@jax.jit
def _sc_sort(score):
    B = score.shape[0]
    mesh = plsc.VectorSubcoreMesh(core_axis_name="c", subcore_axis_name="s")
    f = functools.partial(
        pl.kernel,
        out_type=[jax.ShapeDtypeStruct((B, NOUT), jnp.int32),
                  jax.ShapeDtypeStruct((B, NOUT), jnp.int32)],
        mesh=mesh,
        compiler_params=pltpu.CompilerParams(needs_layout_passes=False),
        scratch_types=[
            pltpu.VMEM((NTOK,), jnp.float32),
            pltpu.VMEM((NG,), jnp.float32),
            pltpu.VMEM((NG,), jnp.int32),
            pltpu.VMEM((NOUT,), jnp.int32),
            pltpu.VMEM((NOUT,), jnp.int32),
        ],
    )(_sc_sort_body)
    return f(score)


def _gather_body(tok_ref, x_ref, o_ref):
    tok = tok_ref[0]
    x = x_ref[0]
    lane = jax.lax.broadcasted_iota(jnp.int32, (NOUT, NTOK), 1)
    row = jax.lax.broadcasted_iota(jnp.int32, (NOUT, 1), 0)
    is_m = row < MERGE
    base = jnp.where(is_m, 64 * (tok >> 4) + 2 * (tok & 15), tok)
    hit = lane == base
    hit = hit | (is_m & ((lane == base + 1) | (lane == base + 32) | (lane == base + 33)))
    w = jnp.where(is_m, jnp.float32(0.25), jnp.float32(1.0))
    s = jnp.where(hit, w, jnp.float32(0.0)).astype(jnp.bfloat16)
    xb = x.astype(jnp.bfloat16)
    o_ref[0] = jax.lax.dot_general(
        s, xb, (((1,), (0,)), ((), ())), preferred_element_type=jnp.float32)


@jax.jit
def _gather_pallas(x_flat, tok3):
    B = x_flat.shape[0]
    return pl.pallas_call(
        _gather_body,
        grid=(B,),
        in_specs=[
            pl.BlockSpec((1, NOUT, 1), lambda b: (b, 0, 0)),
            pl.BlockSpec((1, NTOK, CH), lambda b: (b, 0, 0)),
        ],
        out_specs=pl.BlockSpec((1, NOUT, CH), lambda b: (b, 0, 0)),
        out_shape=jax.ShapeDtypeStruct((B, NOUT, CH), jnp.float32),
    )(tok3, x_flat)


def kernel(x):
    B, C, H, W = x.shape
    xt3 = jnp.transpose(x, (0, 2, 3, 1)).reshape(B, H * W, C)
    smap = _score_pallas(xt3).reshape(B, NTOK)
    token_idx, aligned = _sc_sort(smap)
    x_out = _gather_pallas(xt3, token_idx.reshape(B, NOUT, 1))
    return x_out, aligned
